# baseline (speedup 1.0000x reference)
import jax
import jax.numpy as jnp
from jax.experimental import pallas as pl
from jax.experimental.pallas import tpu as pltpu

TOPK = 2


def _moe_body(x_ref, gw_ref, gp_ref, up_ref, dp_ref, out_ref, w_scr,
              x16_scr):
    e = pl.program_id(0)
    tb = pl.program_id(1)
    BT = w_scr.shape[0] // pl.num_programs(1)
    E = gw_ref.shape[0]
    eids = jax.lax.broadcasted_iota(jnp.int32, (BT, E), 1)

    @pl.when(e == 0)
    def _router():
        xb32 = x_ref[pl.ds(tb * BT, BT), :]
        logits = jax.lax.dot_general(
            xb32, gw_ref[...], (((1,), (1,)), ((), ())),
            preferred_element_type=jnp.float32)
        p = jax.nn.softmax(logits, axis=-1)
        i1 = jnp.argmax(p, axis=-1)
        m1 = jnp.max(p, axis=-1, keepdims=True)
        p2 = jnp.where(eids == i1[:, None], -jnp.inf, p)
        i2 = jnp.argmax(p2, axis=-1)
        m2 = jnp.max(p2, axis=-1, keepdims=True)
        w = (jnp.where(eids == i1[:, None], m1, 0.0)
             + jnp.where(eids == i2[:, None], m2, 0.0)) / (m1 + m2)
        w_scr[pl.ds(tb * BT, BT), :] = w
        x16_scr[pl.ds(tb * BT, BT), :] = xb32.astype(jnp.bfloat16)

    wcol = jnp.sum(w_scr[pl.ds(tb * BT, BT), :] * (eids == e), axis=-1,
                   keepdims=True)

    xb = x16_scr[pl.ds(tb * BT, BT), :]
    g = jnp.dot(xb, gp_ref[0].astype(jnp.bfloat16),
                preferred_element_type=jnp.float32)
    u = jnp.dot(xb, up_ref[0].astype(jnp.bfloat16),
                preferred_element_type=jnp.float32)
    act = (g * jax.nn.sigmoid(g)) * u
    y = jnp.dot(act.astype(jnp.bfloat16), dp_ref[0].astype(jnp.bfloat16),
                preferred_element_type=jnp.float32)
    contrib = y * wcol

    @pl.when(e == 0)
    def _init():
        out_ref[pl.ds(tb * BT, BT), :] = contrib

    @pl.when(e != 0)
    def _acc():
        out_ref[pl.ds(tb * BT, BT), :] += contrib


def kernel(hidden_states, gate_w, gate_proj, up_proj, down_proj):
    b, s, h = hidden_states.shape
    x = hidden_states.reshape(-1, h)
    T = x.shape[0]
    E, H, F = gate_proj.shape
    BT = 2048
    TB = T // BT

    out = pl.pallas_call(
        _moe_body,
        grid=(E, TB),
        in_specs=[
            pl.BlockSpec((T, H), lambda e, tb: (0, 0)),
            pl.BlockSpec((E, H), lambda e, tb: (0, 0)),
            pl.BlockSpec((1, H, F), lambda e, tb: (e, 0, 0)),
            pl.BlockSpec((1, H, F), lambda e, tb: (e, 0, 0)),
            pl.BlockSpec((1, F, H), lambda e, tb: (e, 0, 0)),
        ],
        out_specs=pl.BlockSpec((T, H), lambda e, tb: (0, 0)),
        out_shape=jax.ShapeDtypeStruct((T, H), jnp.float32),
        scratch_shapes=[
            pltpu.VMEM((T, E), jnp.float32),
            pltpu.VMEM((T, H), jnp.bfloat16),
        ],
        compiler_params=pltpu.CompilerParams(
            dimension_semantics=("arbitrary", "arbitrary")),
    )(x, gate_w, gate_proj, up_proj, down_proj)
    return out.reshape(b, s, h)


# speedup vs baseline: 1.0057x; 1.0057x over previous
import jax
import jax.numpy as jnp
from jax.experimental import pallas as pl
from jax.experimental.pallas import tpu as pltpu

TOPK = 2


def _moe_body(x_ref, gw_ref, gp_ref, up_ref, dp_ref, out_ref, w_scr):
    e = pl.program_id(0)
    tb = pl.program_id(1)
    BT = w_scr.shape[0] // pl.num_programs(1)
    E = gw_ref.shape[0]
    eids = jax.lax.broadcasted_iota(jnp.int32, (BT, E), 1)

    @pl.when(e == 0)
    def _router():
        xb32 = x_ref[pl.ds(tb * BT, BT), :]
        logits = jax.lax.dot_general(
            xb32, gw_ref[...], (((1,), (1,)), ((), ())),
            preferred_element_type=jnp.float32)
        p = jax.nn.softmax(logits, axis=-1)
        i1 = jnp.argmax(p, axis=-1)
        m1 = jnp.max(p, axis=-1, keepdims=True)
        p2 = jnp.where(eids == i1[:, None], -jnp.inf, p)
        i2 = jnp.argmax(p2, axis=-1)
        m2 = jnp.max(p2, axis=-1, keepdims=True)
        w = (jnp.where(eids == i1[:, None], m1, 0.0)
             + jnp.where(eids == i2[:, None], m2, 0.0)) / (m1 + m2)
        w_scr[pl.ds(tb * BT, BT), :] = w

    wcol = jnp.sum(w_scr[pl.ds(tb * BT, BT), :] * (eids == e), axis=-1,
                   keepdims=True)

    xb = x_ref[pl.ds(tb * BT, BT), :]
    g = jnp.dot(xb, gp_ref[0], preferred_element_type=jnp.float32)
    u = jnp.dot(xb, up_ref[0], preferred_element_type=jnp.float32)
    act = (g * jax.nn.sigmoid(g)) * u
    y = jnp.dot(act, dp_ref[0], preferred_element_type=jnp.float32)
    contrib = y * wcol

    @pl.when(e == 0)
    def _init():
        out_ref[pl.ds(tb * BT, BT), :] = contrib

    @pl.when(e != 0)
    def _acc():
        out_ref[pl.ds(tb * BT, BT), :] += contrib


def kernel(hidden_states, gate_w, gate_proj, up_proj, down_proj):
    b, s, h = hidden_states.shape
    x = hidden_states.reshape(-1, h)
    T = x.shape[0]
    E, H, F = gate_proj.shape
    BT = 2048
    TB = T // BT

    out = pl.pallas_call(
        _moe_body,
        grid=(E, TB),
        in_specs=[
            pl.BlockSpec((T, H), lambda e, tb: (0, 0)),
            pl.BlockSpec((E, H), lambda e, tb: (0, 0)),
            pl.BlockSpec((1, H, F), lambda e, tb: (e, 0, 0)),
            pl.BlockSpec((1, H, F), lambda e, tb: (e, 0, 0)),
            pl.BlockSpec((1, F, H), lambda e, tb: (e, 0, 0)),
        ],
        out_specs=pl.BlockSpec((T, H), lambda e, tb: (0, 0)),
        out_shape=jax.ShapeDtypeStruct((T, H), jnp.float32),
        scratch_shapes=[
            pltpu.VMEM((T, E), jnp.float32),
        ],
        compiler_params=pltpu.CompilerParams(
            dimension_semantics=("arbitrary", "arbitrary")),
    )(x, gate_w, gate_proj, up_proj, down_proj)
    return out.reshape(b, s, h)
